# baseline (speedup 1.0000x reference)
import jax
import jax.numpy as jnp
from jax import lax
from jax.experimental import pallas as pl
from jax.experimental.pallas import tpu as pltpu
from jax.experimental.pallas import tpu_sc as plsc

L_Q = 2048
L_K = 2048
D = 64
N_EMB = 129
SHIFT = L_K - 1 - (N_EMB - 1) // 2
N_SUB = 32
ROWS_PER_SUB = L_Q // N_SUB
HALF_K = L_K // 2
WIN_ROWS = HALF_K + ROWS_PER_SUB - 1


def _sc_body(table_hbm, out_hbm, table_v, win_v):
    c = lax.axis_index("c")
    s = lax.axis_index("s")
    wid = s * 2 + c
    base = wid * ROWS_PER_SUB

    pltpu.sync_copy(table_hbm, table_v)

    for half in range(2):
        win_lo = half * HALF_K + (L_K - 1 - (ROWS_PER_SUB - 1)) - base

        def build_row(w, carry):
            src = jnp.clip(win_lo + w - SHIFT, 0, N_EMB - 1)
            for q in range(D // 16):
                win_v[w, pl.ds(q * 16, 16)] = table_v[src, pl.ds(q * 16, 16)]
            return carry

        lax.fori_loop(0, WIN_ROWS, build_row, 0)

        def copy_row(r, carry):
            pltpu.sync_copy(
                win_v.at[pl.ds(ROWS_PER_SUB - 1 - r, HALF_K), :],
                out_hbm.at[base + r, pl.ds(half * HALF_K, HALF_K), :],
            )
            return carry

        lax.fori_loop(0, ROWS_PER_SUB, copy_row, 0)


def kernel(length_q, length_k, embeddings_table):
    del length_q, length_k

    call = pl.kernel(
        _sc_body,
        out_type=jax.ShapeDtypeStruct((L_Q, L_K, D), jnp.float32),
        mesh=plsc.VectorSubcoreMesh(core_axis_name="c", subcore_axis_name="s"),
        scratch_types=[
            pltpu.VMEM((N_EMB, D), jnp.float32),
            pltpu.VMEM((WIN_ROWS, D), jnp.float32),
        ],
        compiler_params=pltpu.CompilerParams(use_tc_tiling_on_sc=False),
    )
    return call(embeddings_table)


# speedup vs baseline: 6.0703x; 6.0703x over previous
import jax
import jax.numpy as jnp
from jax import lax
from jax.experimental import pallas as pl
from jax.experimental.pallas import tpu as pltpu
from jax.experimental.pallas import tpu_sc as plsc

L_Q = 2048
L_K = 2048
D = 64
N_EMB = 129
SHIFT = L_K - 1 - (N_EMB - 1) // 2
PAD = 15
PW = N_EMB + 2 * PAD
N_PHASE_PER_SUB = 128 // 32
T_PER_PHASE = 16
WIN_COLS = 128 * (T_PER_PHASE - 1) + L_K
N_GROUP = WIN_COLS // 16
DH = 16


def _sc_body(ttp_hbm, out_hbm, ttp_v, win_v):
    c = lax.axis_index("c")
    s = lax.axis_index("s")
    wid = s * 2 + c
    p0 = wid * N_PHASE_PER_SUB

    pltpu.sync_copy(ttp_hbm, ttp_v)

    def group_fill(dd, q, p, g):
        u = p + g * 16 - SHIFT
        off = q * PW + jnp.clip(u, -PAD, N_EMB - 1) + PAD
        win_v[dd, pl.ds(g * 16, 16)] = ttp_v[pl.ds(off, 16)]

    for dh in range(4):
        def full_build(dd, carry):
            q = dh * DH + dd

            def tile_col(tc, carry2):
                for gg in range(8):
                    group_fill(dd, q, p0, tc * 8 + gg)
                return carry2

            lax.fori_loop(0, N_GROUP // 8, tile_col, 0)
            return carry

        lax.fori_loop(0, DH, full_build, 0)

        for k in range(N_PHASE_PER_SUB):
            p = p0 + k
            if k > 0:
                def band_fix(dd, carry):
                    q = dh * DH + dd
                    g0 = (SHIFT - PAD - p) // 16 - 1
                    for gg in range(12):
                        group_fill(dd, q, p, g0 + gg)
                    return carry

                lax.fori_loop(0, DH, band_fix, 0)

            for t in range(T_PER_PHASE):
                i = L_Q - 1 - p - 128 * t
                pltpu.sync_copy(
                    win_v.at[:, pl.ds(128 * t, L_K)],
                    out_hbm.at[i, pl.ds(dh * DH, DH), :],
                )


def kernel(length_q, length_k, embeddings_table):
    del length_q, length_k
    tt = jnp.swapaxes(embeddings_table, 0, 1)
    ttp = jnp.concatenate(
        [jnp.repeat(tt[:, :1], PAD, axis=1), tt, jnp.repeat(tt[:, -1:], PAD, axis=1)],
        axis=1,
    ).reshape(D * PW)

    call = pl.kernel(
        _sc_body,
        out_type=jax.ShapeDtypeStruct((L_Q, D, L_K), jnp.float32),
        mesh=plsc.VectorSubcoreMesh(core_axis_name="c", subcore_axis_name="s"),
        scratch_types=[
            pltpu.VMEM((D * PW,), jnp.float32),
            pltpu.VMEM((DH, WIN_COLS), jnp.float32),
        ],
    )
    out = call(ttp)
    return jnp.transpose(out, (0, 2, 1))


# speedup vs baseline: 6.1003x; 1.0049x over previous
import jax
import jax.numpy as jnp
from jax import lax
from jax.experimental import pallas as pl
from jax.experimental.pallas import tpu as pltpu
from jax.experimental.pallas import tpu_sc as plsc

L_Q = 2048
L_K = 2048
D = 64
N_EMB = 129
SHIFT = L_K - 1 - (N_EMB - 1) // 2
PAD = 15
PW = N_EMB + 2 * PAD
N_PHASE_PER_SUB = 128 // 32
T_PER_PHASE = 16
WIN_COLS = 128 * (T_PER_PHASE - 1) + L_K
N_GROUP = WIN_COLS // 16
DH = 16


def _sc_body(ttp_hbm, out_hbm, ttp_v, win_v, sem):
    c = lax.axis_index("c")
    s = lax.axis_index("s")
    wid = s * 2 + c
    p0 = wid * N_PHASE_PER_SUB

    pltpu.sync_copy(ttp_hbm, ttp_v)

    def group_fill(dd, q, p, g):
        u = p + g * 16 - SHIFT
        off = q * PW + jnp.clip(u, -PAD, N_EMB - 1) + PAD
        win_v[dd, pl.ds(g * 16, 16)] = ttp_v[pl.ds(off, 16)]

    for dh in range(4):
        def full_build(dd, carry):
            q = dh * DH + dd

            def tile_col(tc, carry2):
                for gg in range(8):
                    group_fill(dd, q, p0, tc * 8 + gg)
                return carry2

            lax.fori_loop(0, N_GROUP // 8, tile_col, 0)
            return carry

        lax.fori_loop(0, DH, full_build, 0)

        for k in range(N_PHASE_PER_SUB):
            p = p0 + k
            if k > 0:
                def band_fix(dd, carry):
                    q = dh * DH + dd
                    g0 = (SHIFT - PAD - p) // 16 - 1
                    for gg in range(12):
                        group_fill(dd, q, p, g0 + gg)
                    return carry

                lax.fori_loop(0, DH, band_fix, 0)

            handles = []
            for t in range(T_PER_PHASE):
                i = L_Q - 1 - p - 128 * t
                handles.append(pltpu.async_copy(
                    win_v.at[:, pl.ds(128 * t, L_K)],
                    out_hbm.at[i, pl.ds(dh * DH, DH), :],
                    sem,
                ))
            for h in handles:
                h.wait()


def kernel(length_q, length_k, embeddings_table):
    del length_q, length_k
    tt = jnp.swapaxes(embeddings_table, 0, 1)
    ttp = jnp.concatenate(
        [jnp.repeat(tt[:, :1], PAD, axis=1), tt, jnp.repeat(tt[:, -1:], PAD, axis=1)],
        axis=1,
    ).reshape(D * PW)

    call = pl.kernel(
        _sc_body,
        out_type=jax.ShapeDtypeStruct((L_Q, D, L_K), jnp.float32),
        mesh=plsc.VectorSubcoreMesh(core_axis_name="c", subcore_axis_name="s"),
        scratch_types=[
            pltpu.VMEM((D * PW,), jnp.float32),
            pltpu.VMEM((DH, WIN_COLS), jnp.float32),
            pltpu.SemaphoreType.DMA,
        ],
    )
    out = call(ttp)
    return jnp.transpose(out, (0, 2, 1))


# speedup vs baseline: 6.9001x; 1.1311x over previous
import jax
import jax.numpy as jnp
from jax import lax
from jax.experimental import pallas as pl
from jax.experimental.pallas import tpu as pltpu
from jax.experimental.pallas import tpu_sc as plsc

L_Q = 2048
L_K = 2048
D = 64
N_EMB = 129
SHIFT = L_K - 1 - (N_EMB - 1) // 2
PAD = 15
PW = N_EMB + 2 * PAD
N_PHASE_PER_SUB = 128 // 32
T_PER_PHASE = 16
WIN_COLS = 128 * (T_PER_PHASE - 1) + L_K
N_GROUP = WIN_COLS // 16
DH = 16


def _sc_body(ttp_hbm, out_hbm, ttp_v, win_v, sem):
    c = lax.axis_index("c")
    s = lax.axis_index("s")
    wid = s * 2 + c
    p0 = wid * N_PHASE_PER_SUB

    pltpu.sync_copy(ttp_hbm, ttp_v)

    def group_fill(dd, q, p, g):
        u = p + g * 16 - SHIFT
        off = q * PW + jnp.clip(u, -PAD, N_EMB - 1) + PAD
        win_v[dd, pl.ds(g * 16, 16)] = ttp_v[pl.ds(off, 16)]

    for dh in range(4):
        def full_build(dd, carry):
            q = dh * DH + dd
            v0 = ttp_v[pl.ds(q * PW, 16)]
            v1 = ttp_v[pl.ds(q * PW + PW - 16, 16)]
            gl = (SHIFT - 15 - p0) // 16 + 1
            nr8 = (N_GROUP - (gl + 12) + 7) // 8
            nl8 = (gl + 7) // 8

            def right_blk(j, carry2):
                g0 = N_GROUP - 8 * (j + 1)
                for gg in range(8):
                    win_v[dd, pl.ds((g0 + gg) * 16, 16)] = v1
                return carry2

            def left_blk(j, carry2):
                for gg in range(8):
                    win_v[dd, pl.ds((8 * j + gg) * 16, 16)] = v0
                return carry2

            lax.fori_loop(0, nr8, right_blk, 0)
            lax.fori_loop(0, nl8, left_blk, 0)
            for gg in range(12):
                group_fill(dd, q, p0, gl + gg)
            return carry

        lax.fori_loop(0, DH, full_build, 0)

        for k in range(N_PHASE_PER_SUB):
            p = p0 + k
            if k > 0:
                def band_fix(dd, carry):
                    q = dh * DH + dd
                    g0 = (SHIFT - PAD - p) // 16 - 1
                    for gg in range(12):
                        group_fill(dd, q, p, g0 + gg)
                    return carry

                lax.fori_loop(0, DH, band_fix, 0)

            handles = []
            for t in range(T_PER_PHASE):
                i = L_Q - 1 - p - 128 * t
                handles.append(pltpu.async_copy(
                    win_v.at[:, pl.ds(128 * t, L_K)],
                    out_hbm.at[i, pl.ds(dh * DH, DH), :],
                    sem,
                ))
            for h in handles:
                h.wait()


def kernel(length_q, length_k, embeddings_table):
    del length_q, length_k
    tt = jnp.swapaxes(embeddings_table, 0, 1)
    ttp = jnp.concatenate(
        [jnp.repeat(tt[:, :1], PAD, axis=1), tt, jnp.repeat(tt[:, -1:], PAD, axis=1)],
        axis=1,
    ).reshape(D * PW)

    call = pl.kernel(
        _sc_body,
        out_type=jax.ShapeDtypeStruct((L_Q, D, L_K), jnp.float32),
        mesh=plsc.VectorSubcoreMesh(core_axis_name="c", subcore_axis_name="s"),
        scratch_types=[
            pltpu.VMEM((D * PW,), jnp.float32),
            pltpu.VMEM((DH, WIN_COLS), jnp.float32),
            pltpu.SemaphoreType.DMA,
        ],
    )
    out = call(ttp)
    return jnp.transpose(out, (0, 2, 1))


# speedup vs baseline: 6.9681x; 1.0098x over previous
import jax
import jax.numpy as jnp
from jax import lax
from jax.experimental import pallas as pl
from jax.experimental.pallas import tpu as pltpu
from jax.experimental.pallas import tpu_sc as plsc

L_Q = 2048
L_K = 2048
D = 64
N_EMB = 129
SHIFT = L_K - 1 - (N_EMB - 1) // 2
PAD = 15
PW = N_EMB + 2 * PAD
N_PHASE_PER_SUB = 128 // 32
T_PER_PHASE = 16
WIN_COLS = 3840
N_GROUP = WIN_COLS // 16
DH = 32
HEAD = 256
CONST_B = 2176


def _sc_body(ttp_hbm, out_hbm, ttp_v, win_v, sem):
    c = lax.axis_index("c")
    s = lax.axis_index("s")
    wid = s * 2 + c
    p0 = wid * N_PHASE_PER_SUB

    def group_fill(dd, p, g):
        u = p + g * 16 - SHIFT
        off = dd * PW + jnp.clip(u, -PAD, N_EMB - 1) + PAD
        win_v[dd, pl.ds(g * 16, 16)] = ttp_v[pl.ds(off, 16)]

    for dh in range(2):
        pltpu.sync_copy(ttp_hbm.at[pl.ds(dh * DH * PW, DH * PW)], ttp_v)

        def full_build(dd, carry):
            v0 = ttp_v[pl.ds(dd * PW, 16)]
            v1 = ttp_v[pl.ds(dd * PW + PW - 16, 16)]
            gl = (SHIFT - 15 - p0) // 16 + 1
            nr8 = (N_GROUP - (gl + 12) + 7) // 8
            nl8 = (gl + 7) // 8

            def right_blk(j, carry2):
                g0 = N_GROUP - 8 * (j + 1)
                for gg in range(8):
                    win_v[dd, pl.ds((g0 + gg) * 16, 16)] = v1
                return carry2

            def left_blk(j, carry2):
                for gg in range(8):
                    win_v[dd, pl.ds((8 * j + gg) * 16, 16)] = v0
                return carry2

            lax.fori_loop(0, nr8, right_blk, 0)
            lax.fori_loop(0, nl8, left_blk, 0)
            for gg in range(12):
                group_fill(dd, p0, gl + gg)
            return carry

        lax.fori_loop(0, DH, full_build, 0)

        for k in range(N_PHASE_PER_SUB):
            p = p0 + k
            if k > 0:
                def band_fix(dd, carry):
                    g0 = (SHIFT - PAD - p) // 16 - 1
                    for gg in range(12):
                        group_fill(dd, p, g0 + gg)
                    return carry

                lax.fori_loop(0, DH, band_fix, 0)

            handles = []
            for t in range(T_PER_PHASE - 1):
                i = L_Q - 1 - p - 128 * t
                handles.append(pltpu.async_copy(
                    win_v.at[:, pl.ds(128 * t, L_K)],
                    out_hbm.at[i, pl.ds(dh * DH, DH), :],
                    sem,
                ))
            i15 = 127 - p
            handles.append(pltpu.async_copy(
                win_v.at[:, pl.ds(1920, HEAD)],
                out_hbm.at[i15, pl.ds(dh * DH, DH), pl.ds(0, HEAD)],
                sem,
            ))
            handles.append(pltpu.async_copy(
                win_v.at[:, pl.ds(CONST_B, WIN_COLS - CONST_B)],
                out_hbm.at[i15, pl.ds(dh * DH, DH), pl.ds(HEAD, WIN_COLS - CONST_B)],
                sem,
            ))
            handles.append(pltpu.async_copy(
                win_v.at[:, pl.ds(CONST_B, L_K - HEAD - (WIN_COLS - CONST_B))],
                out_hbm.at[i15, pl.ds(dh * DH, DH),
                           pl.ds(HEAD + WIN_COLS - CONST_B,
                                 L_K - HEAD - (WIN_COLS - CONST_B))],
                sem,
            ))
            for h in handles:
                h.wait()


def kernel(length_q, length_k, embeddings_table):
    del length_q, length_k
    tt = jnp.swapaxes(embeddings_table, 0, 1)
    ttp = jnp.concatenate(
        [jnp.repeat(tt[:, :1], PAD, axis=1), tt, jnp.repeat(tt[:, -1:], PAD, axis=1)],
        axis=1,
    ).reshape(D * PW)

    call = pl.kernel(
        _sc_body,
        out_type=jax.ShapeDtypeStruct((L_Q, D, L_K), jnp.float32),
        mesh=plsc.VectorSubcoreMesh(core_axis_name="c", subcore_axis_name="s"),
        scratch_types=[
            pltpu.VMEM((DH * PW,), jnp.float32),
            pltpu.VMEM((DH, WIN_COLS), jnp.float32),
            pltpu.SemaphoreType.DMA,
        ],
    )
    out = call(ttp)
    return jnp.transpose(out, (0, 2, 1))


# speedup vs baseline: 7.0280x; 1.0086x over previous
import jax
import jax.numpy as jnp
from jax import lax
from jax.experimental import pallas as pl
from jax.experimental.pallas import tpu as pltpu
from jax.experimental.pallas import tpu_sc as plsc

L_Q = 2048
L_K = 2048
D = 64
N_EMB = 129
SHIFT = L_K - 1 - (N_EMB - 1) // 2
PAD = 15
PW = N_EMB + 2 * PAD
N_PHASE_PER_SUB = 128 // 32
T_PER_PHASE = 16
WIN_COLS = 3840
N_GROUP = WIN_COLS // 16
DH = 32
HEAD = 256
CONST_B = 2176


def _sc_body(ttp_hbm, out_hbm, ttp_v, win_v, sem):
    c = lax.axis_index("c")
    s = lax.axis_index("s")
    wid = s * 2 + c
    p0 = wid * N_PHASE_PER_SUB

    def group_fill(dd, p, g):
        u = p + g * 16 - SHIFT
        off = dd * PW + jnp.clip(u, -PAD, N_EMB - 1) + PAD
        win_v[dd, pl.ds(g * 16, 16)] = ttp_v[pl.ds(off, 16)]

    def fire_late(p, dh, handles):
        i14 = L_Q - 1 - p - 128 * 14
        handles.append(pltpu.async_copy(
            win_v.at[:, pl.ds(128 * 14, L_K)],
            out_hbm.at[i14, pl.ds(dh * DH, DH), :],
            sem,
        ))
        i15 = 127 - p
        handles.append(pltpu.async_copy(
            win_v.at[:, pl.ds(1920, HEAD)],
            out_hbm.at[i15, pl.ds(dh * DH, DH), pl.ds(0, HEAD)],
            sem,
        ))
        handles.append(pltpu.async_copy(
            win_v.at[:, pl.ds(CONST_B, WIN_COLS - CONST_B)],
            out_hbm.at[i15, pl.ds(dh * DH, DH), pl.ds(HEAD, WIN_COLS - CONST_B)],
            sem,
        ))
        handles.append(pltpu.async_copy(
            win_v.at[:, pl.ds(CONST_B, L_K - HEAD - (WIN_COLS - CONST_B))],
            out_hbm.at[i15, pl.ds(dh * DH, DH),
                       pl.ds(HEAD + WIN_COLS - CONST_B,
                             L_K - HEAD - (WIN_COLS - CONST_B))],
            sem,
        ))

    def fire_early(p, dh, handles):
        for t in range(T_PER_PHASE - 2):
            i = L_Q - 1 - p - 128 * t
            handles.append(pltpu.async_copy(
                win_v.at[:, pl.ds(128 * t, L_K)],
                out_hbm.at[i, pl.ds(dh * DH, DH), :],
                sem,
            ))

    for dh in range(2):
        pltpu.sync_copy(ttp_hbm.at[pl.ds(dh * DH * PW, DH * PW)], ttp_v)

        def build_hi(dd, carry):
            v0 = ttp_v[pl.ds(dd * PW, 16)]
            v1 = ttp_v[pl.ds(dd * PW + PW - 16, 16)]
            gl = (SHIFT - 15 - p0) // 16 + 1
            nr8 = (N_GROUP - (gl + 12) + 7) // 8

            def right_blk(j, carry2):
                g0 = N_GROUP - 8 * (j + 1)
                for gg in range(8):
                    win_v[dd, pl.ds((g0 + gg) * 16, 16)] = v1
                return carry2

            lax.fori_loop(0, nr8, right_blk, 0)
            for gg in range(12):
                win_v[dd, pl.ds((gl - 12 + gg) * 16, 16)] = v0
            for gg in range(12):
                group_fill(dd, p0, gl + gg)
            return carry

        lax.fori_loop(0, DH, build_hi, 0)

        handles = []
        fire_late(p0, dh, handles)

        def build_lo(dd, carry):
            v0 = ttp_v[pl.ds(dd * PW, 16)]

            def left_blk(j, carry2):
                for gg in range(8):
                    win_v[dd, pl.ds((8 * j + gg) * 16, 16)] = v0
                return carry2

            lax.fori_loop(0, 112 // 8, left_blk, 0)
            return carry

        lax.fori_loop(0, DH, build_lo, 0)
        fire_early(p0, dh, handles)
        for h in handles:
            h.wait()

        for k in range(1, N_PHASE_PER_SUB):
            p = p0 + k
            if True:
                def band_fix(dd, carry):
                    g0 = (SHIFT - PAD - p) // 16 - 1
                    for gg in range(12):
                        group_fill(dd, p, g0 + gg)
                    return carry

                lax.fori_loop(0, DH, band_fix, 0)

            handles = []
            fire_late(p, dh, handles)
            fire_early(p, dh, handles)
            for h in handles:
                h.wait()


def kernel(length_q, length_k, embeddings_table):
    del length_q, length_k
    tt = jnp.swapaxes(embeddings_table, 0, 1)
    ttp = jnp.concatenate(
        [jnp.repeat(tt[:, :1], PAD, axis=1), tt, jnp.repeat(tt[:, -1:], PAD, axis=1)],
        axis=1,
    ).reshape(D * PW)

    call = pl.kernel(
        _sc_body,
        out_type=jax.ShapeDtypeStruct((L_Q, D, L_K), jnp.float32),
        mesh=plsc.VectorSubcoreMesh(core_axis_name="c", subcore_axis_name="s"),
        scratch_types=[
            pltpu.VMEM((DH * PW,), jnp.float32),
            pltpu.VMEM((DH, WIN_COLS), jnp.float32),
            pltpu.SemaphoreType.DMA,
        ],
    )
    out = call(ttp)
    return jnp.transpose(out, (0, 2, 1))
